# baseline (speedup 1.0000x reference)
import functools

import jax
import jax.numpy as jnp
from jax import lax
from jax.experimental import pallas as pl
from jax.experimental.pallas import tpu as pltpu
from jax.experimental.pallas import tpu_sc as plsc

_EPS = 1e-12
_LANES = 16
_CHUNK = 8


def _rsqrt16(x):
    i = lax.bitcast_convert_type(x, jnp.int32)
    i = jnp.int32(0x5F3759DF) - lax.shift_right_logical(i, 1)
    y = lax.bitcast_convert_type(i, jnp.float32)
    for _ in range(3):
        y = y * (1.5 - 0.5 * x * y * y)
    return y


def _allsum(v):
    dnums = lax.GatherDimensionNumbers(
        offset_dims=(), collapsed_slice_dims=(0,), start_index_map=(0,))
    for st in (1, 2, 4, 8):
        idx = lax.iota(jnp.int32, _LANES) ^ st
        v = v + lax.gather(v, idx[:, None], dnums, slice_sizes=(1,),
                           mode=lax.GatherScatterMode.PROMISE_IN_BOUNDS)
    return v


def _make_sc_kernel(n_tok, hid, seq_len):
    info = plsc.get_sparse_core_info()
    nw = info.num_cores * info.num_subcores
    per_w = n_tok // nw
    n_chunks = per_w // _CHUNK
    n_sl = hid // _LANES
    mesh = plsc.VectorSubcoreMesh(core_axis_name="c", subcore_axis_name="s")

    @functools.partial(
        pl.kernel,
        out_type=jax.ShapeDtypeStruct((n_tok, hid), jnp.float32),
        mesh=mesh,
        scratch_types=[
            pltpu.VMEM((2 * per_w,), jnp.int32),
            pltpu.VMEM((per_w,), jnp.int32),
            [pltpu.VMEM((2 * _CHUNK, hid), jnp.float32)] * 2,
            [pltpu.VMEM((_CHUNK, hid), jnp.float32)] * 2,
            [pltpu.VMEM((_CHUNK, hid), jnp.float32)] * 2,
            pltpu.VMEM((_CHUNK, hid), jnp.float32),
            pltpu.VMEM((hid,), jnp.float32),
            pltpu.VMEM((hid,), jnp.float32),
            [pltpu.SemaphoreType.DMA] * 2,
        ],
    )
    def k(widx_h, cidx_h, word_h, comb_h, pos_h, lnw_h, lnb_h, out_h,
          iw, ic, bufw, bufc, bufp, obuf, w_v, b_v, sems):
        wid = lax.axis_index("s") * info.num_cores + lax.axis_index("c")
        base = wid * per_w
        s_base = base % seq_len
        pltpu.sync_copy(lnw_h, w_v)
        pltpu.sync_copy(lnb_h, b_v)
        pltpu.sync_copy(widx_h.at[pl.ds(2 * base, 2 * per_w)], iw)
        pltpu.sync_copy(cidx_h.at[pl.ds(base, per_w)], ic)

        def prefetch(kk, slot):
            if True:
                return
            sem = sems[slot]
            pltpu.async_copy(
                word_h.at[iw.at[pl.ds(kk * 2 * _CHUNK, 2 * _CHUNK)]],
                bufw[slot], sem)
            pltpu.async_copy(
                comb_h.at[ic.at[pl.ds(kk * _CHUNK, _CHUNK)]],
                bufc[slot], sem)
            pltpu.async_copy(
                pos_h.at[pl.ds(s_base + kk * _CHUNK, _CHUNK)],
                bufp[slot], sem)

        def wait_gathers(slot):
            if True:
                return
            sem = sems[slot]
            pltpu.make_async_copy(
                word_h.at[iw.at[pl.ds(0, 2 * _CHUNK)]], bufw[slot],
                sem).wait()
            pltpu.make_async_copy(
                comb_h.at[ic.at[pl.ds(0, _CHUNK)]], bufc[slot], sem).wait()
            pltpu.make_async_copy(
                pos_h.at[pl.ds(0, _CHUNK)], bufp[slot], sem).wait()

        def compute(kk, slot):
            bw, bc, bp = bufw[slot], bufc[slot], bufp[slot]

            def tok_body(t, tc):
                acc1 = jnp.zeros((_LANES,), jnp.float32)
                acc2 = jnp.zeros((_LANES,), jnp.float32)
                for d in range(n_sl):
                    sl = pl.ds(d * _LANES, _LANES)
                    x = (bw[2 * t, sl] + bw[2 * t + 1, sl] + bc[t, sl]
                         + bp[t, sl])
                    bw[2 * t, sl] = x
                    acc1 = acc1 + x
                    acc2 = acc2 + x * x
                mu = _allsum(acc1) * (1.0 / hid)
                ex2 = _allsum(acc2) * (1.0 / hid)
                inv = _rsqrt16(ex2 - mu * mu + _EPS)
                for d in range(n_sl):
                    sl = pl.ds(d * _LANES, _LANES)
                    obuf[t, sl] = (bw[2 * t, sl] - mu) * inv * w_v[sl] + b_v[sl]
                return tc

            lax.fori_loop(0, _CHUNK, tok_body, 0)
            pltpu.sync_copy(obuf, out_h.at[pl.ds(base + kk * _CHUNK, _CHUNK)])

        prefetch(0, 0)

        def pair_body(k2, carry):
            for half in (0, 1):
                kk = k2 * 2 + half

                @pl.when(kk + 1 < n_chunks)
                def _():
                    prefetch(kk + 1, 1 - half)

                wait_gathers(half)
                compute(kk, half)
            return carry

        lax.fori_loop(0, n_chunks // 2, pair_body, 0)

    return k


def kernel(input_ids, pos_ids, graph_rel, token_type_ids, word_emb, label_emb,
           pos_table, type_emb, ln_w, ln_b):
    b, s = input_ids.shape
    hid = word_emb.shape[1]
    n_tok = b * s
    widx = jnp.stack(
        [input_ids.reshape(-1), pos_ids.reshape(-1)], axis=-1
    ).reshape(-1).astype(jnp.int32)
    comb_idx = (graph_rel.reshape(-1) * type_emb.shape[0]
                + token_type_ids.reshape(-1)).astype(jnp.int32)
    comb = (label_emb[:, None, :] + type_emb[None, :, :]).reshape(-1, hid)
    k = _make_sc_kernel(n_tok, hid, s)
    out = k(widx, comb_idx, word_emb, comb, pos_table, ln_w, ln_b)
    return out.reshape(b, s, hid)


# speedup vs baseline: 1.6985x; 1.6985x over previous
import functools

import jax
import jax.numpy as jnp
from jax import lax
from jax.experimental import pallas as pl
from jax.experimental.pallas import tpu as pltpu
from jax.experimental.pallas import tpu_sc as plsc

_EPS = 1e-12
_LANES = 16
_CHUNK = 16
_TBLK = 256


def _make_sc_gather_sum(n_tok, hid):
    info = plsc.get_sparse_core_info()
    nw = info.num_cores * info.num_subcores
    per_w = n_tok // nw
    n_chunks = per_w // _CHUNK
    n_sl = hid // _LANES
    mesh = plsc.VectorSubcoreMesh(core_axis_name="c", subcore_axis_name="s")

    @functools.partial(
        pl.kernel,
        out_type=jax.ShapeDtypeStruct((n_tok, hid), jnp.float32),
        mesh=mesh,
        scratch_types=[
            pltpu.VMEM((2 * per_w,), jnp.int32),
            [pltpu.VMEM((2 * _CHUNK, hid), jnp.float32)] * 2,
            pltpu.VMEM((_CHUNK, hid), jnp.float32),
            [pltpu.SemaphoreType.DMA] * 2,
        ],
    )
    def k(widx_h, word_h, out_h, iw, bufw, obuf, sems):
        wid = lax.axis_index("s") * info.num_cores + lax.axis_index("c")
        base = wid * per_w
        pltpu.sync_copy(widx_h.at[pl.ds(2 * base, 2 * per_w)], iw)

        def prefetch(kk, slot):
            pltpu.async_copy(
                word_h.at[iw.at[pl.ds(kk * 2 * _CHUNK, 2 * _CHUNK)]],
                bufw[slot], sems[slot])

        def wait_gather(slot):
            pltpu.make_async_copy(
                word_h.at[iw.at[pl.ds(0, 2 * _CHUNK)]], bufw[slot],
                sems[slot]).wait()

        def compute(kk, slot):
            bw = bufw[slot]

            def tok_body(t, tc):
                for d in range(n_sl):
                    sl = pl.ds(d * _LANES, _LANES)
                    obuf[t, sl] = bw[2 * t, sl] + bw[2 * t + 1, sl]
                return tc

            lax.fori_loop(0, _CHUNK, tok_body, 0)
            pltpu.sync_copy(obuf, out_h.at[pl.ds(base + kk * _CHUNK, _CHUNK)])

        prefetch(0, 0)

        def pair_body(k2, carry):
            for half in (0, 1):
                kk = k2 * 2 + half

                @pl.when(kk + 1 < n_chunks)
                def _():
                    prefetch(kk + 1, 1 - half)

                wait_gather(half)
                compute(kk, half)
            return carry

        lax.fori_loop(0, n_chunks // 2, pair_body, 0)

    return k


def _tc_dense_body(g_ref, idx_ref, comb_ref, pos_ref, w_ref, b_ref, o_ref):
    idx = idx_ref[0, 0, :]
    n_comb = comb_ref.shape[0]
    oh = (idx[:, None] == lax.broadcasted_iota(jnp.int32, (_TBLK, n_comb), 1)
          ).astype(jnp.float32)
    crows = jnp.dot(oh, comb_ref[...], preferred_element_type=jnp.float32)
    x = g_ref[...] + pos_ref[...] + crows
    mu = jnp.mean(x, axis=-1, keepdims=True)
    var = jnp.mean(jnp.square(x - mu), axis=-1, keepdims=True)
    o_ref[...] = ((x - mu) * lax.rsqrt(var + _EPS) * w_ref[...]
                  + b_ref[...])


def kernel(input_ids, pos_ids, graph_rel, token_type_ids, word_emb, label_emb,
           pos_table, type_emb, ln_w, ln_b):
    b, s = input_ids.shape
    hid = word_emb.shape[1]
    n_tok = b * s
    widx = jnp.stack(
        [input_ids.reshape(-1), pos_ids.reshape(-1)], axis=-1
    ).reshape(-1).astype(jnp.int32)
    n_types = type_emb.shape[0]
    comb = (label_emb[:, None, :] + type_emb[None, :, :]).reshape(-1, hid)
    n_comb = comb.shape[0]
    comb_idx = (graph_rel.reshape(-1) * n_types
                + token_type_ids.reshape(-1)).astype(jnp.int32)

    g = _make_sc_gather_sum(n_tok, hid)(widx, word_emb)

    n_blk = n_tok // _TBLK
    n_sblk = s // _TBLK
    idx3 = comb_idx.reshape(n_blk, 1, _TBLK)
    out = pl.pallas_call(
        _tc_dense_body,
        grid=(n_blk,),
        in_specs=[
            pl.BlockSpec((_TBLK, hid), lambda i: (i, 0)),
            pl.BlockSpec((1, 1, _TBLK), lambda i: (i, 0, 0)),
            pl.BlockSpec((n_comb, hid), lambda i: (0, 0)),
            pl.BlockSpec((_TBLK, hid), lambda i: (i % n_sblk, 0)),
            pl.BlockSpec((1, hid), lambda i: (0, 0)),
            pl.BlockSpec((1, hid), lambda i: (0, 0)),
        ],
        out_specs=pl.BlockSpec((_TBLK, hid), lambda i: (i, 0)),
        out_shape=jax.ShapeDtypeStruct((n_tok, hid), jnp.float32),
    )(g, idx3, comb, pos_table, ln_w.reshape(1, hid), ln_b.reshape(1, hid))
    return out.reshape(b, s, hid)


# speedup vs baseline: 1.7285x; 1.0177x over previous
import functools

import jax
import jax.numpy as jnp
from jax import lax
from jax.experimental import pallas as pl
from jax.experimental.pallas import tpu as pltpu
from jax.experimental.pallas import tpu_sc as plsc

_EPS = 1e-12
_LANES = 16
_CHUNK = 8
_NBUF = 4
_TBLK = 256


def _make_sc_gather_sum(n_tok, hid):
    info = plsc.get_sparse_core_info()
    nw = info.num_cores * info.num_subcores
    per_w = n_tok // nw
    n_chunks = per_w // _CHUNK
    n_sl = hid // _LANES
    mesh = plsc.VectorSubcoreMesh(core_axis_name="c", subcore_axis_name="s")

    @functools.partial(
        pl.kernel,
        out_type=jax.ShapeDtypeStruct((n_tok, hid), jnp.float32),
        mesh=mesh,
        scratch_types=[
            pltpu.VMEM((2 * per_w,), jnp.int32),
            [pltpu.VMEM((2 * _CHUNK, hid), jnp.float32)] * _NBUF,
            [pltpu.VMEM((_CHUNK, hid), jnp.float32)] * 2,
            [pltpu.SemaphoreType.DMA] * _NBUF,
            [pltpu.SemaphoreType.DMA] * 2,
        ],
    )
    def k(widx_h, word_h, out_h, iw, bufw, obufs, sems, osems):
        wid = lax.axis_index("s") * info.num_cores + lax.axis_index("c")
        base = wid * per_w
        pltpu.sync_copy(widx_h.at[pl.ds(2 * base, 2 * per_w)], iw)

        def prefetch(kk, slot):
            pltpu.async_copy(
                word_h.at[iw.at[pl.ds(kk * 2 * _CHUNK, 2 * _CHUNK)]],
                bufw[slot], sems[slot])

        def wait_gather(slot):
            pltpu.make_async_copy(
                word_h.at[iw.at[pl.ds(0, 2 * _CHUNK)]], bufw[slot],
                sems[slot]).wait()

        def out_wait(par):
            pltpu.make_async_copy(
                obufs[par], out_h.at[pl.ds(base, _CHUNK)], osems[par]).wait()

        def compute(kk, slot, par):
            bw = bufw[slot]
            ob = obufs[par]

            def tok_body(t, tc):
                for d in range(n_sl):
                    sl = pl.ds(d * _LANES, _LANES)
                    ob[t, sl] = bw[2 * t, sl] + bw[2 * t + 1, sl]
                return tc

            lax.fori_loop(0, _CHUNK, tok_body, 0)
            pltpu.async_copy(
                ob, out_h.at[pl.ds(base + kk * _CHUNK, _CHUNK)], osems[par])

        for j in range(_NBUF - 1):
            prefetch(j, j)

        def ring_body(k4, carry):
            for j in range(_NBUF):
                kk = k4 * _NBUF + j
                wait_gather(j)

                @pl.when(kk + _NBUF - 1 < n_chunks)
                def _():
                    prefetch(kk + _NBUF - 1, (j + _NBUF - 1) % _NBUF)

                par = j % 2

                @pl.when(kk >= 2)
                def _():
                    out_wait(par)

                compute(kk, j, par)
            return carry

        lax.fori_loop(0, n_chunks // _NBUF, ring_body, 0)
        out_wait(0)
        out_wait(1)

    return k


def _tc_dense_body(g_ref, idx_ref, comb_ref, pos_ref, w_ref, b_ref, o_ref):
    idx = idx_ref[0, 0, :]
    n_comb = comb_ref.shape[0]
    oh = (idx[:, None] == lax.broadcasted_iota(jnp.int32, (_TBLK, n_comb), 1)
          ).astype(jnp.float32)
    crows = jnp.dot(oh, comb_ref[...], preferred_element_type=jnp.float32,
                    precision=lax.Precision.HIGHEST)
    x = g_ref[...] + pos_ref[...] + crows
    mu = jnp.mean(x, axis=-1, keepdims=True)
    var = jnp.mean(jnp.square(x - mu), axis=-1, keepdims=True)
    o_ref[...] = ((x - mu) * lax.rsqrt(var + _EPS) * w_ref[...]
                  + b_ref[...])


def kernel(input_ids, pos_ids, graph_rel, token_type_ids, word_emb, label_emb,
           pos_table, type_emb, ln_w, ln_b):
    b, s = input_ids.shape
    hid = word_emb.shape[1]
    n_tok = b * s
    widx = jnp.stack(
        [input_ids.reshape(-1), pos_ids.reshape(-1)], axis=-1
    ).reshape(-1).astype(jnp.int32)
    n_types = type_emb.shape[0]
    comb = (label_emb[:, None, :] + type_emb[None, :, :]).reshape(-1, hid)
    n_comb = comb.shape[0]
    comb_idx = (graph_rel.reshape(-1) * n_types
                + token_type_ids.reshape(-1)).astype(jnp.int32)

    g = _make_sc_gather_sum(n_tok, hid)(widx, word_emb)

    n_blk = n_tok // _TBLK
    n_sblk = s // _TBLK
    idx3 = comb_idx.reshape(n_blk, 1, _TBLK)

    def tok_blk(i):
        return (i % b) * n_sblk + i // b

    out = pl.pallas_call(
        _tc_dense_body,
        grid=(n_blk,),
        in_specs=[
            pl.BlockSpec((_TBLK, hid), lambda i: (tok_blk(i), 0)),
            pl.BlockSpec((1, 1, _TBLK), lambda i: (tok_blk(i), 0, 0)),
            pl.BlockSpec((n_comb, hid), lambda i: (0, 0)),
            pl.BlockSpec((_TBLK, hid), lambda i: (i // b, 0)),
            pl.BlockSpec((1, hid), lambda i: (0, 0)),
            pl.BlockSpec((1, hid), lambda i: (0, 0)),
        ],
        out_specs=pl.BlockSpec((_TBLK, hid), lambda i: (tok_blk(i), 0)),
        out_shape=jax.ShapeDtypeStruct((n_tok, hid), jnp.float32),
    )(g, idx3, comb, pos_table, ln_w.reshape(1, hid), ln_b.reshape(1, hid))
    return out.reshape(b, s, hid)


# speedup vs baseline: 2.5106x; 1.4525x over previous
import functools

import jax
import jax.numpy as jnp
from jax import lax
from jax.experimental import pallas as pl
from jax.experimental.pallas import tpu as pltpu
from jax.experimental.pallas import tpu_sc as plsc

_EPS = 1e-12
_LANES = 16
_CHUNK = 8
_NBUF = 4
_TBLK = 256


def _make_sc_gather_sum(n_tok, hid):
    info = plsc.get_sparse_core_info()
    nw = info.num_cores * info.num_subcores
    per_w = n_tok // nw
    n_chunks = per_w // _CHUNK
    n_sl = hid // _LANES
    mesh = plsc.VectorSubcoreMesh(core_axis_name="c", subcore_axis_name="s")

    @functools.partial(
        pl.kernel,
        out_type=jax.ShapeDtypeStruct((n_tok, hid), jnp.float32),
        mesh=mesh,
        scratch_types=[
            pltpu.VMEM((per_w,), jnp.int32),
            pltpu.VMEM((per_w,), jnp.int32),
            [pltpu.VMEM((_CHUNK, hid), jnp.float32)] * _NBUF,
            [pltpu.VMEM((_CHUNK, hid), jnp.float32)] * _NBUF,
            [pltpu.VMEM((_CHUNK, hid), jnp.float32)] * 2,
            [pltpu.SemaphoreType.DMA] * _NBUF,
            [pltpu.SemaphoreType.DMA] * 2,
        ],
    )
    def k(aidx_h, bidx_h, word_h, out_h, ia, ib, bufa, bufb, obufs, sems,
          osems):
        wid = lax.axis_index("s") * info.num_cores + lax.axis_index("c")
        base = wid * per_w
        pltpu.sync_copy(aidx_h.at[pl.ds(base, per_w)], ia)
        pltpu.sync_copy(bidx_h.at[pl.ds(base, per_w)], ib)

        def prefetch(kk, slot):
            isl = pl.ds(kk * _CHUNK, _CHUNK)
            pltpu.async_copy(word_h.at[ia.at[isl]], bufa[slot], sems[slot])
            pltpu.async_copy(word_h.at[ib.at[isl]], bufb[slot], sems[slot])

        def wait_gather(slot):
            isl = pl.ds(0, _CHUNK)
            pltpu.make_async_copy(
                word_h.at[ia.at[isl]], bufa[slot], sems[slot]).wait()
            pltpu.make_async_copy(
                word_h.at[ib.at[isl]], bufb[slot], sems[slot]).wait()

        def out_wait(par):
            pltpu.make_async_copy(
                obufs[par], out_h.at[pl.ds(base, _CHUNK)], osems[par]).wait()

        def compute(kk, slot, par):
            ba, bb = bufa[slot], bufb[slot]
            ob = obufs[par]

            def tok_body(t, tc):
                for d in range(n_sl):
                    sl = pl.ds(d * _LANES, _LANES)
                    ob[t, sl] = ba[t, sl] + bb[t, sl]
                return tc

            lax.fori_loop(0, _CHUNK, tok_body, 0)
            pltpu.async_copy(
                ob, out_h.at[pl.ds(base + kk * _CHUNK, _CHUNK)], osems[par])

        for j in range(_NBUF - 1):
            prefetch(j, j)

        def ring_body(k4, carry):
            for j in range(_NBUF):
                kk = k4 * _NBUF + j
                wait_gather(j)

                @pl.when(kk + _NBUF - 1 < n_chunks)
                def _():
                    prefetch(kk + _NBUF - 1, (j + _NBUF - 1) % _NBUF)

                par = j % 2

                @pl.when(kk >= 2)
                def _():
                    out_wait(par)

                compute(kk, j, par)
            return carry

        lax.fori_loop(0, n_chunks // _NBUF, ring_body, 0)
        out_wait(0)
        out_wait(1)

    return k


def _tc_dense_body(g_ref, idx_ref, comb_ref, pos_ref, w_ref, b_ref, o_ref):
    idx = idx_ref[0, 0, :]
    n_comb = comb_ref.shape[0]
    oh = (idx[:, None] == lax.broadcasted_iota(jnp.int32, (_TBLK, n_comb), 1)
          ).astype(jnp.float32)
    crows = jnp.dot(oh, comb_ref[...], preferred_element_type=jnp.float32,
                    precision=lax.Precision.HIGHEST)
    x = g_ref[...] + pos_ref[...] + crows
    mu = jnp.mean(x, axis=-1, keepdims=True)
    var = jnp.mean(jnp.square(x - mu), axis=-1, keepdims=True)
    o_ref[...] = ((x - mu) * lax.rsqrt(var + _EPS) * w_ref[...]
                  + b_ref[...])


def kernel(input_ids, pos_ids, graph_rel, token_type_ids, word_emb, label_emb,
           pos_table, type_emb, ln_w, ln_b):
    b, s = input_ids.shape
    hid = word_emb.shape[1]
    n_tok = b * s
    aidx = input_ids.reshape(-1).astype(jnp.int32)
    bidx = pos_ids.reshape(-1).astype(jnp.int32)
    n_types = type_emb.shape[0]
    comb = (label_emb[:, None, :] + type_emb[None, :, :]).reshape(-1, hid)
    n_comb = comb.shape[0]
    comb_idx = (graph_rel.reshape(-1) * n_types
                + token_type_ids.reshape(-1)).astype(jnp.int32)

    g = _make_sc_gather_sum(n_tok, hid)(aidx, bidx, word_emb)

    n_blk = n_tok // _TBLK
    n_sblk = s // _TBLK
    idx3 = comb_idx.reshape(n_blk, 1, _TBLK)

    def tok_blk(i):
        return (i % b) * n_sblk + i // b

    out = pl.pallas_call(
        _tc_dense_body,
        grid=(n_blk,),
        in_specs=[
            pl.BlockSpec((_TBLK, hid), lambda i: (tok_blk(i), 0)),
            pl.BlockSpec((1, 1, _TBLK), lambda i: (tok_blk(i), 0, 0)),
            pl.BlockSpec((n_comb, hid), lambda i: (0, 0)),
            pl.BlockSpec((_TBLK, hid), lambda i: (i // b, 0)),
            pl.BlockSpec((1, hid), lambda i: (0, 0)),
            pl.BlockSpec((1, hid), lambda i: (0, 0)),
        ],
        out_specs=pl.BlockSpec((_TBLK, hid), lambda i: (tok_blk(i), 0)),
        out_shape=jax.ShapeDtypeStruct((n_tok, hid), jnp.float32),
    )(g, idx3, comb, pos_table, ln_w.reshape(1, hid), ln_b.reshape(1, hid))
    return out.reshape(b, s, hid)


# speedup vs baseline: 2.7362x; 1.0899x over previous
import functools

import jax
import jax.numpy as jnp
from jax import lax
from jax.experimental import pallas as pl
from jax.experimental.pallas import tpu as pltpu
from jax.experimental.pallas import tpu_sc as plsc

_EPS = 1e-12
_LANES = 16
_CHUNK = 8
_NBUF = 4
_TBLK = 256


def _make_sc_gather_sum(n_tok, hid):
    info = plsc.get_sparse_core_info()
    nw = info.num_cores * info.num_subcores
    per_w = n_tok // nw
    n_chunks = per_w // _CHUNK
    n_sl = hid // _LANES
    mesh = plsc.VectorSubcoreMesh(core_axis_name="c", subcore_axis_name="s")

    @functools.partial(
        pl.kernel,
        out_type=jax.ShapeDtypeStruct((n_tok, hid), jnp.float32),
        mesh=mesh,
        scratch_types=[
            pltpu.VMEM((per_w,), jnp.int32),
            pltpu.VMEM((per_w,), jnp.int32),
            [pltpu.VMEM((_CHUNK, hid), jnp.float32)] * _NBUF,
            [pltpu.VMEM((_CHUNK, hid), jnp.float32)] * _NBUF,
            [pltpu.VMEM((_CHUNK, hid), jnp.float32)] * 2,
            [pltpu.SemaphoreType.DMA] * _NBUF,
            [pltpu.SemaphoreType.DMA] * 2,
        ],
    )
    def k(aidx_h, bidx_h, word_h, out_h, ia, ib, bufa, bufb, obufs, sems,
          osems):
        wid = lax.axis_index("s") * info.num_cores + lax.axis_index("c")
        base = wid * per_w
        pltpu.sync_copy(aidx_h.at[pl.ds(base, per_w)], ia)
        pltpu.sync_copy(bidx_h.at[pl.ds(base, per_w)], ib)

        def prefetch(kk, slot):
            isl = pl.ds(kk * _CHUNK, _CHUNK)
            pltpu.async_copy(word_h.at[ia.at[isl]], bufa[slot], sems[slot])
            pltpu.async_copy(word_h.at[ib.at[isl]], bufb[slot], sems[slot])

        def wait_gather(slot):
            isl = pl.ds(0, _CHUNK)
            pltpu.make_async_copy(
                word_h.at[ia.at[isl]], bufa[slot], sems[slot]).wait()
            pltpu.make_async_copy(
                word_h.at[ib.at[isl]], bufb[slot], sems[slot]).wait()

        def out_wait(par):
            pltpu.make_async_copy(
                obufs[par], out_h.at[pl.ds(base, _CHUNK)], osems[par]).wait()

        def compute(kk, slot, par):
            ba, bb = bufa[slot], bufb[slot]
            ob = obufs[par]

            def tok_body(t, tc):
                for d in range(n_sl):
                    sl = pl.ds(d * _LANES, _LANES)
                    ob[t, sl] = ba[t, sl] + bb[t, sl]
                return tc

            lax.fori_loop(0, _CHUNK, tok_body, 0)
            pltpu.async_copy(
                ob, out_h.at[pl.ds(base + kk * _CHUNK, _CHUNK)], osems[par])

        for j in range(_NBUF - 1):
            prefetch(j, j)

        def ring_body(k4, carry):
            for j in range(_NBUF):
                kk = k4 * _NBUF + j
                wait_gather(j)

                @pl.when(kk + _NBUF - 1 < n_chunks)
                def _():
                    prefetch(kk + _NBUF - 1, (j + _NBUF - 1) % _NBUF)

                par = j % 2

                @pl.when(kk >= 2)
                def _():
                    out_wait(par)

                compute(kk, j, par)
            return carry

        lax.fori_loop(0, n_chunks // _NBUF, ring_body, 0)
        out_wait(0)
        out_wait(1)

    return k


def _tc_dense_body(g_ref, idx_ref, comb_ref, pos_ref, w_ref, b_ref, o_ref):
    idx = idx_ref[0, 0, :]
    n_comb = comb_ref.shape[0]
    oh = (idx[:, None] == lax.broadcasted_iota(jnp.int32, (_TBLK, n_comb), 1)
          ).astype(jnp.float32)
    crows = jnp.dot(oh, comb_ref[...], preferred_element_type=jnp.float32)
    x = g_ref[...] + pos_ref[...] + crows
    mu = jnp.mean(x, axis=-1, keepdims=True)
    var = jnp.mean(jnp.square(x - mu), axis=-1, keepdims=True)
    o_ref[...] = ((x - mu) * lax.rsqrt(var + _EPS) * w_ref[...]
                  + b_ref[...])


def kernel(input_ids, pos_ids, graph_rel, token_type_ids, word_emb, label_emb,
           pos_table, type_emb, ln_w, ln_b):
    b, s = input_ids.shape
    hid = word_emb.shape[1]
    n_tok = b * s
    aidx = input_ids.reshape(-1).astype(jnp.int32)
    bidx = pos_ids.reshape(-1).astype(jnp.int32)
    n_types = type_emb.shape[0]
    comb = (label_emb[:, None, :] + type_emb[None, :, :]).reshape(-1, hid)
    n_comb = comb.shape[0]
    comb_idx = (graph_rel.reshape(-1) * n_types
                + token_type_ids.reshape(-1)).astype(jnp.int32)

    g = _make_sc_gather_sum(n_tok, hid)(aidx, bidx, word_emb)

    n_blk = n_tok // _TBLK
    n_sblk = s // _TBLK
    idx3 = comb_idx.reshape(n_blk, 1, _TBLK)

    def tok_blk(i):
        return (i % b) * n_sblk + i // b

    out = pl.pallas_call(
        _tc_dense_body,
        grid=(n_blk,),
        in_specs=[
            pl.BlockSpec((_TBLK, hid), lambda i: (tok_blk(i), 0)),
            pl.BlockSpec((1, 1, _TBLK), lambda i: (tok_blk(i), 0, 0)),
            pl.BlockSpec((n_comb, hid), lambda i: (0, 0)),
            pl.BlockSpec((_TBLK, hid), lambda i: (i // b, 0)),
            pl.BlockSpec((1, hid), lambda i: (0, 0)),
            pl.BlockSpec((1, hid), lambda i: (0, 0)),
        ],
        out_specs=pl.BlockSpec((_TBLK, hid), lambda i: (tok_blk(i), 0)),
        out_shape=jax.ShapeDtypeStruct((n_tok, hid), jnp.float32),
    )(g, idx3, comb, pos_table, ln_w.reshape(1, hid), ln_b.reshape(1, hid))
    return out.reshape(b, s, hid)
